# baseline (speedup 1.0000x reference)
import functools

import jax
import jax.numpy as jnp
import numpy as np
from jax import lax
from jax.experimental import pallas as pl
from jax.experimental.pallas import tpu as pltpu
from jax.experimental.pallas import tpu_sc as plsc

STRIDES = [8.0, 16.0, 32.0]
IMAGE_SIZE = 640
FEATURE_SIZES = [int(IMAGE_SIZE / s) for s in STRIDES]
MAX_N_LABELS = [16384, 8192, 4096]
ANCHOR_THRESHOLD = 4.0
HALF_MAX = 65504.0
_ANCHOR_W = [[12.0, 19.0, 40.0], [36.0, 76.0, 72.0], [142.0, 192.0, 459.0]]
_ANCHOR_H = [[16.0, 36.0, 28.0], [75.0, 55.0, 146.0], [110.0, 243.0, 401.0]]
ANCHORS_NP = [np.stack([np.array(_ANCHOR_W[i], dtype=np.float32),
                        np.array(_ANCHOR_H[i], dtype=np.float32)], axis=1)
              / np.float32(STRIDES[i]) for i in range(3)]

B = 64
N = 1000
NPAD = 1024
NB = B * N * 3
CAPS = [32768, 16384, 8192]
NT = 32


def _tc_a_kernel(labs_ref, sbits_ref, vmw_ref, th_ref):
    c = labs_ref[0]
    u1 = labs_ref[1]
    u2 = labs_ref[2]
    u3 = labs_ref[3]
    u4 = labs_ref[4]
    g = 0.5
    for s in range(3):
        fs = float(FEATURE_SIZES[s])
        x = u1 * fs
        y = u2 * fs
        w = u3 * fs
        h = u4 * fs
        w0 = w[0:1, :]
        h0 = h[0:1, :]
        for a in range(3):
            aw = float(ANCHORS_NP[s][a, 0])
            ah = float(ANCHORS_NP[s][a, 1])
            rw = w0 / aw
            rh = h0 / ah
            worse = jnp.maximum(jnp.maximum(rw, 1.0 / rw),
                                jnp.maximum(rh, 1.0 / rh))
            worse = jnp.where(worse != 0.0, worse, HALF_MAX)
            mask = worse < ANCHOR_THRESHOLD
            maskb = jnp.broadcast_to(mask, (B, NPAD))
            aid = float(a + 1)
            score = ((((c + x) + y) + w) + h) + aid
            score = jnp.where(maskb, score, 0.0)
            bx = jnp.where(maskb, x, 0.0)
            by = jnp.where(maskb, y, 0.0)
            ibx = jnp.where(bx != 0.0, fs - bx, 0.0)
            iby = jnp.where(by != 0.0, fs - by, 0.0)
            xm = (jnp.mod(bx, 1.0) < g) & (bx > 1.0)
            ym = (jnp.mod(by, 1.0) < g) & (by > 1.0)
            ixm = (jnp.mod(ibx, 1.0) < g) & (ibx > 1.0)
            iym = (jnp.mod(iby, 1.0) < g) & (iby > 1.0)
            vm = (xm.astype(jnp.int32) | (ym.astype(jnp.int32) << 1)
                  | (ixm.astype(jnp.int32) << 2) | (iym.astype(jnp.int32) << 3))
            cnt = (1 + xm.astype(jnp.int32) + ym.astype(jnp.int32)
                   + ixm.astype(jnp.int32) + iym.astype(jnp.int32))
            sbits_ref[s * 3 + a] = lax.bitcast_convert_type(score, jnp.int32)
            vmw_ref[s * 3 + a] = vm | (cnt << 8)

    ths = []
    for s in range(3):
        bits3 = [sbits_ref[s * 3 + a] for a in range(3)]
        cnt3 = [(vmw_ref[s * 3 + a] >> 8) for a in range(3)]
        K = MAX_N_LABELS[s]

        def step(i, carry, bits3=bits3, cnt3=cnt3, K=K):
            p, kr = carry
            pos = 30 - i
            want = (p << 1) | 1
            c1 = jnp.int32(0)
            for bb, cc in zip(bits3, cnt3):
                m = lax.shift_right_logical(bb, pos) == want
                c1 = c1 + jnp.sum(jnp.where(m, cc, 0))
            take_hi = c1 >= kr
            p = jnp.where(take_hi, want, p << 1)
            kr = jnp.where(take_hi, kr, kr - c1)
            return p, kr

        p, _ = lax.fori_loop(0, 31, step, (jnp.int32(0), jnp.int32(K)))
        ths.append(p)
    rowi = lax.broadcasted_iota(jnp.int32, (8, 128), 0)
    th_ref[...] = jnp.where(rowi == 0, ths[0],
                            jnp.where(rowi == 1, ths[1], ths[2]))


def _run_tc_a(labs_t, interpret=False):
    return pl.pallas_call(
        _tc_a_kernel,
        out_shape=(
            jax.ShapeDtypeStruct((9, B, NPAD), jnp.int32),
            jax.ShapeDtypeStruct((9, B, NPAD), jnp.int32),
            jax.ShapeDtypeStruct((8, 128), jnp.int32),
        ),
        interpret=interpret,
    )(labs_t)


def _partner_rows(x, m):
    r = x.shape[0]
    y = x.reshape(r // (2 * m), 2, m, 128)
    y = jnp.concatenate([y[:, 1:2], y[:, 0:1]], axis=1)
    return y.reshape(r, 128)


def _partner_lanes(x, j):
    lane = lax.broadcasted_iota(jnp.int32, x.shape, 1)
    lo = pltpu.roll(x, 128 - j, 1)
    hi = pltpu.roll(x, j, 1)
    return jnp.where((lane & j) == 0, lo, hi)


def _bitonic_pair(k1, k2, n):
    rows = n // 128
    e_row = lax.broadcasted_iota(jnp.int32, (rows, 128), 0)
    e_lane = lax.broadcasted_iota(jnp.int32, (rows, 128), 1)
    logn = n.bit_length() - 1
    for kl in range(1, logn + 1):
        k = 1 << kl
        if k < 128:
            asc = (e_lane & k) == 0
        elif k < n:
            asc = (e_row & (k // 128)) == 0
        else:
            asc = (e_lane & 0) == 0
        for jl in range(kl - 1, -1, -1):
            j = 1 << jl
            if j < 128:
                p1 = _partner_lanes(k1, j)
                p2 = _partner_lanes(k2, j)
                upper = (e_lane & j) != 0
            else:
                p1 = _partner_rows(k1, j // 128)
                p2 = _partner_rows(k2, j // 128)
                upper = (e_row & (j // 128)) != 0
            less = (p1 < k1) | ((p1 == k1) & (p2 < k2))
            take = less == (asc != upper)
            k1 = jnp.where(take, p1, k1)
            k2 = jnp.where(take, p2, k2)
    return k1, k2


def _tc_c_kernel(s0, i0, s1, i1, s2, i2, os0, oi0, os1, oi1, os2, oi2):
    for (si, ii, oo_s, oo_i, cap) in ((s0, i0, os0, oi0, CAPS[0]),
                                      (s1, i1, os1, oi1, CAPS[1]),
                                      (s2, i2, os2, oi2, CAPS[2])):
        k1 = ~si[...]
        k2 = ii[...]
        k1, k2 = _bitonic_pair(k1, k2, cap)
        oo_s[...] = ~k1
        oo_i[...] = k2


def _run_tc_c(comp, interpret=False):
    args = []
    shapes = []
    for s in range(3):
        cs, ci = comp[s]
        args += [cs.reshape(CAPS[s] // 128, 128), ci.reshape(CAPS[s] // 128, 128)]
        shapes += [jax.ShapeDtypeStruct((CAPS[s] // 128, 128), jnp.int32)] * 2
    outs = pl.pallas_call(
        _tc_c_kernel,
        out_shape=tuple(shapes),
        interpret=interpret,
    )(*args)
    return [(outs[2 * s].reshape(-1), outs[2 * s + 1].reshape(-1))
            for s in range(3)]


def _sc_mesh():
    return plsc.VectorSubcoreMesh(core_axis_name="c", subcore_axis_name="s")
_BIG = jnp.int32(0x7FFFFFFF)


def _sc_b_body(sb_hbm, vw_hbm, th_hbm,
               oS0, oI0, oS1, oI1, oS2, oI2,
               sbuf, vbuf, thv, bufS, bufI, sem):
    del sem
    wid = lax.axis_index("s") * 2 + lax.axis_index("c")
    lanes = lax.iota(jnp.int32, 16)
    outs = ((oS0, oI0), (oS1, oI1), (oS2, oI2))
    for s in range(3):
        cap = CAPS[s] // NT
        pltpu.sync_copy(th_hbm.at[pl.ds(s * 128, 16)], thv)
        tvec = thv[...]
        pltpu.sync_copy(sb_hbm.at[s, wid], sbuf)
        pltpu.sync_copy(vw_hbm.at[s, wid], vbuf)

        def pad_body(j, _, bufS=bufS, bufI=bufI):
            off = pl.multiple_of(j * 16, 16)
            bufS[pl.ds(off, 16)] = jnp.full((16,), -1, jnp.int32)
            bufI[pl.ds(off, 16)] = jnp.full((16,), _BIG, jnp.int32)
            return 0

        lax.fori_loop(0, cap // 16, pad_body, 0)

        def body(i, cur, s=s, cap=cap, tvec=tvec):
            off = pl.multiple_of(i * 16, 16)
            sv = sbuf[pl.ds(off, 16)]
            vv = vbuf[pl.ds(off, 16)]
            sel = (sv >= tvec) & (sv > 0)
            posg = (wid * 6144 + i * 16) + lanes
            a = lax.shift_right_logical(posg, 16)
            b = lax.shift_right_logical(posg, 10) & 63
            n = posg & 1023
            bidx = (b * 1000 + n) * 3 + a
            for v in range(5):
                if v == 0:
                    mv = sel
                else:
                    mv = sel & ((lax.shift_right_logical(vv, v - 1) & 1) == 1)
                mi = mv.astype(jnp.int32)
                cs = plsc.cumsum(mi)
                pos = (cur + cs) - mi
                okm = mv & (pos < cap)
                plsc.store_scatter(bufS, [pos], sv, mask=okm)
                plsc.store_scatter(bufI, [pos], v * NB + bidx, mask=okm)
                cur = cur + plsc.cummax(lax.rev(cs, (0,)))
            return cur

        lax.fori_loop(0, 6144 // 16, body, jnp.zeros((16,), jnp.int32))
        oS, oI = outs[s]
        pltpu.sync_copy(bufS.at[pl.ds(0, cap)], oS.at[pl.ds(wid * cap, cap)])
        pltpu.sync_copy(bufI.at[pl.ds(0, cap)], oI.at[pl.ds(wid * cap, cap)])


def _run_sc_b(sbits, vmw, th):
    sb = sbits.reshape(3, NT, 6144)
    vw = vmw.reshape(3, NT, 6144)
    thf = th.reshape(1024)
    f = pl.kernel(
        _sc_b_body,
        out_type=tuple(jax.ShapeDtypeStruct((CAPS[s // 2],), jnp.int32)
                       for s in range(6)),
        mesh=_sc_mesh(),
        scratch_types=[
            pltpu.VMEM((6144,), jnp.int32),
            pltpu.VMEM((6144,), jnp.int32),
            pltpu.VMEM((16,), jnp.int32),
            pltpu.VMEM((1024,), jnp.int32),
            pltpu.VMEM((1024,), jnp.int32),
            pltpu.SemaphoreType.DMA,
        ],
    )
    o = f(sb, vw, thf)
    return [(o[0], o[1]), (o[2], o[3]), (o[4], o[5])]


def _sc_d_body(S0, I0, S1, I1, S2, I2, lab_hbm,
               oa0, oy0, ox0, ot00, ot01, ot02, ot03, ot04,
               oa1, oy1, ox1, ot10, ot11, ot12, ot13, ot14,
               oa2, oy2, ox2, ot20, ot21, ot22, ot23, ot24,
               sbuf, ibuf, idx2d, u0b, u1b, u2b, u3b, u4b,
               ab, yb, xb, t0b, t1b, t2b, t3b, t4b, sem):
    wid = lax.axis_index("s") * 2 + lax.axis_index("c")
    ins = ((S0, I0), (S1, I1), (S2, I2))
    outs = ((oa0, oy0, ox0, ot00, ot01, ot02, ot03, ot04),
            (oa1, oy1, ox1, ot10, ot11, ot12, ot13, ot14),
            (oa2, oy2, ox2, ot20, ot21, ot22, ot23, ot24))
    ubs = (u0b, u1b, u2b, u3b, u4b)
    for s in range(3):
        K = MAX_N_LABELS[s]
        cap = K // NT
        fs = float(FEATURE_SIZES[s])
        Sin, Iin = ins[s]
        pltpu.sync_copy(Sin.at[pl.ds(wid * cap, cap)], sbuf.at[pl.ds(0, cap)])
        pltpu.sync_copy(Iin.at[pl.ds(wid * cap, cap)], ibuf.at[pl.ds(0, cap)])

        def rowidx_body(j, acc):
            off = pl.multiple_of(j * 16, 16)
            iv = ibuf[pl.ds(off, 16)]
            sv = sbuf[pl.ds(off, 16)]
            nz = sv > 0
            ivz = jnp.where(nz, iv, 0)
            v = ivz // NB
            base = ivz - v * NB
            bb = base // 3000
            r1 = base - bb * 3000
            nn = r1 // 3
            row = bb * 1000 + nn
            idx2d[pl.ds(off, 16)] = jnp.where(nz, row, 0)
            return acc

        lax.fori_loop(0, cap // 16, rowidx_body, jnp.int32(0))
        for c in range(5):
            for j2 in range(cap // 128):
                pltpu.async_copy(
                    lab_hbm.at[c].at[idx2d.at[pl.ds(j2 * 128, 128)]],
                    ubs[c].at[pl.ds(j2 * 128, 128)], sem).wait()

        def out_body(j, acc, s=s, fs=fs):
            off = pl.multiple_of(j * 16, 16)
            iv = ibuf[pl.ds(off, 16)]
            sv = sbuf[pl.ds(off, 16)]
            nz = sv > 0
            ivz = jnp.where(nz, iv, 0)
            v = ivz // NB
            base = ivz - v * NB
            bb = base // 3000
            r1 = base - bb * 3000
            nn = r1 // 3
            aa = r1 - nn * 3
            zf = jnp.float32(0.0)
            c = jnp.where(nz, u0b[pl.ds(off, 16)], zf)
            x = jnp.where(nz, u1b[pl.ds(off, 16)] * fs, zf)
            y = jnp.where(nz, u2b[pl.ds(off, 16)] * fs, zf)
            w = jnp.where(nz, u3b[pl.ds(off, 16)] * fs, zf)
            h = jnp.where(nz, u4b[pl.ds(off, 16)] * fs, zf)
            oxv = jnp.where(v == 1, 0.5, jnp.where(v == 3, -0.5, 0.0)).astype(jnp.float32)
            oyv = jnp.where(v == 2, 0.5, jnp.where(v == 4, -0.5, 0.0)).astype(jnp.float32)
            oxv = jnp.where(nz, oxv, zf)
            oyv = jnp.where(nz, oyv, zf)
            xi = jnp.where(x != 0.0, (x - oxv).astype(jnp.int32), 0)
            yi = jnp.where(y != 0.0, (y - oyv).astype(jnp.int32), 0)
            fsi = int(fs)
            ab[pl.ds(off, 16)] = jnp.minimum(jnp.maximum(jnp.where(nz, aa, 0), 0), 4)
            yb[pl.ds(off, 16)] = jnp.minimum(jnp.maximum(yi, 0), fsi - 1)
            xb[pl.ds(off, 16)] = jnp.minimum(jnp.maximum(xi, 0), fsi - 1)
            t0b[pl.ds(off, 16)] = c
            t1b[pl.ds(off, 16)] = x - xi.astype(jnp.float32)
            t2b[pl.ds(off, 16)] = y - yi.astype(jnp.float32)
            t3b[pl.ds(off, 16)] = w
            t4b[pl.ds(off, 16)] = h
            return acc

        lax.fori_loop(0, cap // 16, out_body, jnp.int32(0))
        oo = outs[s]
        for buf, dst in zip((ab, yb, xb, t0b, t1b, t2b, t3b, t4b), oo):
            pltpu.sync_copy(buf.at[pl.ds(0, cap)], dst.at[pl.ds(wid * cap, cap)])


def _run_sc_d(sorted_comp, lab_planes):
    ins = []
    for s in range(3):
        ins += [sorted_comp[s][0], sorted_comp[s][1]]
    shapes = []
    for s in range(3):
        K = MAX_N_LABELS[s]
        shapes += [jax.ShapeDtypeStruct((K,), jnp.int32)] * 3 \
            + [jax.ShapeDtypeStruct((K,), jnp.float32)] * 5
    f = pl.kernel(
        _sc_d_body,
        out_type=tuple(shapes),
        mesh=_sc_mesh(),
        scratch_types=[
            pltpu.VMEM((512,), jnp.int32),
            pltpu.VMEM((512,), jnp.int32),
            pltpu.VMEM((512,), jnp.int32),
            pltpu.VMEM((512,), jnp.float32),
            pltpu.VMEM((512,), jnp.float32),
            pltpu.VMEM((512,), jnp.float32),
            pltpu.VMEM((512,), jnp.float32),
            pltpu.VMEM((512,), jnp.float32),
            pltpu.VMEM((512,), jnp.int32),
            pltpu.VMEM((512,), jnp.int32),
            pltpu.VMEM((512,), jnp.int32),
            pltpu.VMEM((512,), jnp.float32),
            pltpu.VMEM((512,), jnp.float32),
            pltpu.VMEM((512,), jnp.float32),
            pltpu.VMEM((512,), jnp.float32),
            pltpu.VMEM((512,), jnp.float32),
            pltpu.SemaphoreType.DMA,
        ],
    )
    o = f(*ins, lab_planes)
    res = []
    for s in range(3):
        g = o[8 * s: 8 * s + 8]
        tb = jnp.stack(g[3:8], axis=-1)
        res.append((g[0], g[1], g[2], tb))
    return res


def _emu_sc_b(sbits, vmw, th):
    out = []
    sb = sbits.reshape(3, 3 * B * NPAD)
    vw = vmw.reshape(3, 3 * B * NPAD)
    pos0 = jnp.arange(3 * B * NPAD, dtype=jnp.int32)
    a = pos0 >> 16
    bb = (pos0 >> 10) & 63
    nn = pos0 & 1023
    bidx = (bb * 1000 + nn) * 3 + a
    for s in range(3):
        t = th[s, 0]
        bits = sb[s]
        vm = vw[s]
        sel = (bits >= t) & (bits > 0)
        masks = [sel]
        for v in range(1, 5):
            masks.append(sel & (((vm >> (v - 1)) & 1) == 1))
        m = jnp.stack(masks, 0).reshape(-1)
        vals_i = jnp.stack([v * NB + bidx for v in range(5)], 0).reshape(-1)
        vals_s = jnp.stack([bits] * 5, 0).reshape(-1)
        mi = m.astype(jnp.int32)
        pos = jnp.cumsum(mi) - 1
        pos = jnp.where(m & (pos < CAPS[s]), pos, CAPS[s])
        outS = jnp.full((CAPS[s] + 1,), -1, jnp.int32)
        outI = jnp.full((CAPS[s] + 1,), 0x7FFFFFFF, jnp.int32)
        outS = outS.at[pos].set(vals_s, mode="drop")
        outI = outI.at[pos].set(vals_i, mode="drop")
        out.append((outS[:CAPS[s]], outI[:CAPS[s]]))
    return out


def _emu_sc_d(sorted_comp, labels_flat):
    outs = []
    for s in range(3):
        K = MAX_N_LABELS[s]
        fs = float(FEATURE_SIZES[s])
        sb, ci = sorted_comp[s]
        sb = sb[:K]
        ci = ci[:K]
        nz = sb > 0
        v = jnp.where(nz, ci // NB, 0)
        base = jnp.where(nz, ci % NB, 0)
        bb = base // 3000
        r1 = base - bb * 3000
        nn = r1 // 3
        aa = r1 - nn * 3
        rows = labels_flat[bb * 1000 + nn]
        c = jnp.where(nz, rows[:, 0], 0.0)
        x = jnp.where(nz, rows[:, 1] * fs, 0.0)
        y = jnp.where(nz, rows[:, 2] * fs, 0.0)
        w = jnp.where(nz, rows[:, 3] * fs, 0.0)
        h = jnp.where(nz, rows[:, 4] * fs, 0.0)
        ox = jnp.where(v == 1, 0.5, jnp.where(v == 3, -0.5, 0.0))
        oy = jnp.where(v == 2, 0.5, jnp.where(v == 4, -0.5, 0.0))
        ox = jnp.where(nz, ox, 0.0)
        oy = jnp.where(nz, oy, 0.0)
        xi = jnp.where(x != 0.0, (x - ox).astype(jnp.int32), 0)
        yi = jnp.where(y != 0.0, (y - oy).astype(jnp.int32), 0)
        x_ind = jnp.clip(xi, 0, FEATURE_SIZES[s] - 1)
        y_ind = jnp.clip(yi, 0, FEATURE_SIZES[s] - 1)
        anchor = jnp.where(nz, aa, 0).astype(jnp.int32)
        tb = jnp.stack([c, x - xi.astype(jnp.float32),
                        y - yi.astype(jnp.float32), w, h], axis=-1)
        outs.append((anchor, y_ind, x_ind, tb))
    return outs


_USE_SC = False
_USE_SC_D = False
_INTERP = False


def kernel(real_labels):
    labs_t = jnp.pad(jnp.transpose(real_labels, (2, 0, 1)),
                     ((0, 0), (0, 0), (0, NPAD - N)))
    sbits, vmw, th = _run_tc_a(labs_t, interpret=_INTERP)
    if _USE_SC:
        comp = _run_sc_b(sbits, vmw, th.reshape(1024))
    else:
        comp = _emu_sc_b(sbits, vmw, th)
    sorted_comp = _run_tc_c(comp, interpret=_INTERP)
    labels_flat = real_labels.reshape(B * N, 5)
    if _USE_SC and _USE_SC_D:
        outs = _run_sc_d(sorted_comp, labels_flat.T)
    else:
        outs = _emu_sc_d(sorted_comp, labels_flat)
    return tuple(outs)


# speedup vs baseline: 7.1511x; 7.1511x over previous
import functools

import jax
import jax.numpy as jnp
import numpy as np
from jax import lax
from jax.experimental import pallas as pl
from jax.experimental.pallas import tpu as pltpu
from jax.experimental.pallas import tpu_sc as plsc

STRIDES = [8.0, 16.0, 32.0]
IMAGE_SIZE = 640
FEATURE_SIZES = [int(IMAGE_SIZE / s) for s in STRIDES]
MAX_N_LABELS = [16384, 8192, 4096]
ANCHOR_THRESHOLD = 4.0
HALF_MAX = 65504.0
_ANCHOR_W = [[12.0, 19.0, 40.0], [36.0, 76.0, 72.0], [142.0, 192.0, 459.0]]
_ANCHOR_H = [[16.0, 36.0, 28.0], [75.0, 55.0, 146.0], [110.0, 243.0, 401.0]]
ANCHORS_NP = [np.stack([np.array(_ANCHOR_W[i], dtype=np.float32),
                        np.array(_ANCHOR_H[i], dtype=np.float32)], axis=1)
              / np.float32(STRIDES[i]) for i in range(3)]

B = 64
N = 1000
NPAD = 1024
NB = B * N * 3
CAPS = [32768, 16384, 8192]
NT = 32


def _tc_a_kernel(labs_ref, sbits_ref, vmw_ref, th_ref):
    c = labs_ref[0]
    u1 = labs_ref[1]
    u2 = labs_ref[2]
    u3 = labs_ref[3]
    u4 = labs_ref[4]
    g = 0.5
    for s in range(3):
        fs = float(FEATURE_SIZES[s])
        x = u1 * fs
        y = u2 * fs
        w = u3 * fs
        h = u4 * fs
        w0 = w[0:1, :]
        h0 = h[0:1, :]
        for a in range(3):
            aw = float(ANCHORS_NP[s][a, 0])
            ah = float(ANCHORS_NP[s][a, 1])
            rw = w0 / aw
            rh = h0 / ah
            worse = jnp.maximum(jnp.maximum(rw, 1.0 / rw),
                                jnp.maximum(rh, 1.0 / rh))
            worse = jnp.where(worse != 0.0, worse, HALF_MAX)
            mask = worse < ANCHOR_THRESHOLD
            maskb = jnp.broadcast_to(mask, (B, NPAD))
            aid = float(a + 1)
            score = ((((c + x) + y) + w) + h) + aid
            score = jnp.where(maskb, score, 0.0)
            bx = jnp.where(maskb, x, 0.0)
            by = jnp.where(maskb, y, 0.0)
            ibx = jnp.where(bx != 0.0, fs - bx, 0.0)
            iby = jnp.where(by != 0.0, fs - by, 0.0)
            xm = (jnp.mod(bx, 1.0) < g) & (bx > 1.0)
            ym = (jnp.mod(by, 1.0) < g) & (by > 1.0)
            ixm = (jnp.mod(ibx, 1.0) < g) & (ibx > 1.0)
            iym = (jnp.mod(iby, 1.0) < g) & (iby > 1.0)
            vm = (xm.astype(jnp.int32) | (ym.astype(jnp.int32) << 1)
                  | (ixm.astype(jnp.int32) << 2) | (iym.astype(jnp.int32) << 3))
            cnt = (1 + xm.astype(jnp.int32) + ym.astype(jnp.int32)
                   + ixm.astype(jnp.int32) + iym.astype(jnp.int32))
            sbits_ref[s * 3 + a] = lax.bitcast_convert_type(score, jnp.int32)
            vmw_ref[s * 3 + a] = vm | (cnt << 8)

    ths = []
    for s in range(3):
        bits3 = [sbits_ref[s * 3 + a] for a in range(3)]
        cnt3 = [(vmw_ref[s * 3 + a] >> 8) for a in range(3)]
        K = MAX_N_LABELS[s]

        def step(i, carry, bits3=bits3, cnt3=cnt3, K=K):
            p, kr = carry
            pos = 30 - i
            want = (p << 1) | 1
            c1 = jnp.int32(0)
            for bb, cc in zip(bits3, cnt3):
                m = lax.shift_right_logical(bb, pos) == want
                c1 = c1 + jnp.sum(jnp.where(m, cc, 0))
            take_hi = c1 >= kr
            p = jnp.where(take_hi, want, p << 1)
            kr = jnp.where(take_hi, kr, kr - c1)
            return p, kr

        p, _ = lax.fori_loop(0, 31, step, (jnp.int32(0), jnp.int32(K)))
        ths.append(p)
    rowi = lax.broadcasted_iota(jnp.int32, (8, 128), 0)
    th_ref[...] = jnp.where(rowi == 0, ths[0],
                            jnp.where(rowi == 1, ths[1], ths[2]))


def _run_tc_a(labs_t, interpret=False):
    return pl.pallas_call(
        _tc_a_kernel,
        out_shape=(
            jax.ShapeDtypeStruct((9, B, NPAD), jnp.int32),
            jax.ShapeDtypeStruct((9, B, NPAD), jnp.int32),
            jax.ShapeDtypeStruct((8, 128), jnp.int32),
        ),
        interpret=interpret,
    )(labs_t)


def _partner_rows(x, m):
    r = x.shape[0]
    y = x.reshape(r // (2 * m), 2, m, 128)
    y = jnp.concatenate([y[:, 1:2], y[:, 0:1]], axis=1)
    return y.reshape(r, 128)


def _partner_lanes(x, j):
    lane = lax.broadcasted_iota(jnp.int32, x.shape, 1)
    lo = pltpu.roll(x, 128 - j, 1)
    hi = pltpu.roll(x, j, 1)
    return jnp.where((lane & j) == 0, lo, hi)


def _bitonic_pair(k1, k2, n):
    rows = n // 128
    e_row = lax.broadcasted_iota(jnp.int32, (rows, 128), 0)
    e_lane = lax.broadcasted_iota(jnp.int32, (rows, 128), 1)
    logn = n.bit_length() - 1
    for kl in range(1, logn + 1):
        k = 1 << kl
        if k < 128:
            asc = (e_lane & k) == 0
        elif k < n:
            asc = (e_row & (k // 128)) == 0
        else:
            asc = (e_lane & 0) == 0
        for jl in range(kl - 1, -1, -1):
            j = 1 << jl
            if j < 128:
                p1 = _partner_lanes(k1, j)
                p2 = _partner_lanes(k2, j)
                upper = (e_lane & j) != 0
            else:
                p1 = _partner_rows(k1, j // 128)
                p2 = _partner_rows(k2, j // 128)
                upper = (e_row & (j // 128)) != 0
            less = (p1 < k1) | ((p1 == k1) & (p2 < k2))
            take = less == (asc != upper)
            k1 = jnp.where(take, p1, k1)
            k2 = jnp.where(take, p2, k2)
    return k1, k2


def _tc_c_kernel(s0, i0, s1, i1, s2, i2, os0, oi0, os1, oi1, os2, oi2):
    for (si, ii, oo_s, oo_i, cap) in ((s0, i0, os0, oi0, CAPS[0]),
                                      (s1, i1, os1, oi1, CAPS[1]),
                                      (s2, i2, os2, oi2, CAPS[2])):
        k1 = ~si[...]
        k2 = ii[...]
        k1, k2 = _bitonic_pair(k1, k2, cap)
        oo_s[...] = ~k1
        oo_i[...] = k2


def _run_tc_c(comp, interpret=False):
    args = []
    shapes = []
    for s in range(3):
        cs, ci = comp[s]
        args += [cs.reshape(CAPS[s] // 128, 128), ci.reshape(CAPS[s] // 128, 128)]
        shapes += [jax.ShapeDtypeStruct((CAPS[s] // 128, 128), jnp.int32)] * 2
    outs = pl.pallas_call(
        _tc_c_kernel,
        out_shape=tuple(shapes),
        interpret=interpret,
    )(*args)
    return [(outs[2 * s].reshape(-1), outs[2 * s + 1].reshape(-1))
            for s in range(3)]


def _sc_mesh():
    return plsc.VectorSubcoreMesh(core_axis_name="c", subcore_axis_name="s")
_BIG = jnp.int32(0x7FFFFFFF)


def _sc_b_body(sb_hbm, vw_hbm, th_hbm,
               oS0, oI0, oS1, oI1, oS2, oI2,
               sbuf, vbuf, thv, bufS, bufI, sem):
    del sem
    wid = lax.axis_index("s") * 2 + lax.axis_index("c")
    lanes = lax.iota(jnp.int32, 16)
    outs = ((oS0, oI0), (oS1, oI1), (oS2, oI2))
    for s in range(3):
        cap = CAPS[s] // NT
        pltpu.sync_copy(th_hbm.at[pl.ds(s * 128, 16)], thv)
        tvec = thv[...]
        pltpu.sync_copy(sb_hbm.at[s, wid], sbuf)
        pltpu.sync_copy(vw_hbm.at[s, wid], vbuf)

        def pad_body(j, _, bufS=bufS, bufI=bufI):
            off = pl.multiple_of(j * 16, 16)
            bufS[pl.ds(off, 16)] = jnp.full((16,), -1, jnp.int32)
            bufI[pl.ds(off, 16)] = jnp.full((16,), _BIG, jnp.int32)
            return 0

        lax.fori_loop(0, cap // 16, pad_body, 0)

        def body(i, cur, s=s, cap=cap, tvec=tvec):
            off = pl.multiple_of(i * 16, 16)
            sv = sbuf[pl.ds(off, 16)]
            vv = vbuf[pl.ds(off, 16)]
            sel = (sv >= tvec) & (sv > 0)
            posg = (wid * 6144 + i * 16) + lanes
            a = lax.shift_right_logical(posg, 16)
            b = lax.shift_right_logical(posg, 10) & 63
            n = posg & 1023
            bidx = (b * 1000 + n) * 3 + a
            for v in range(5):
                if v == 0:
                    mv = sel
                else:
                    mv = sel & ((lax.shift_right_logical(vv, v - 1) & 1) == 1)
                mi = mv.astype(jnp.int32)
                cs = plsc.cumsum(mi)
                pos = (cur + cs) - mi
                okm = mv & (pos < cap)
                plsc.store_scatter(bufS, [pos], sv, mask=okm)
                plsc.store_scatter(bufI, [pos], v * NB + bidx, mask=okm)
                cur = cur + plsc.cummax(lax.rev(cs, (0,)))
            return cur

        lax.fori_loop(0, 6144 // 16, body, jnp.zeros((16,), jnp.int32))
        oS, oI = outs[s]
        pltpu.sync_copy(bufS.at[pl.ds(0, cap)], oS.at[pl.ds(wid * cap, cap)])
        pltpu.sync_copy(bufI.at[pl.ds(0, cap)], oI.at[pl.ds(wid * cap, cap)])


def _run_sc_b(sbits, vmw, th):
    sb = sbits.reshape(3, NT, 6144)
    vw = vmw.reshape(3, NT, 6144)
    thf = th.reshape(1024)
    f = pl.kernel(
        _sc_b_body,
        out_type=tuple(jax.ShapeDtypeStruct((CAPS[s // 2],), jnp.int32)
                       for s in range(6)),
        mesh=_sc_mesh(),
        scratch_types=[
            pltpu.VMEM((6144,), jnp.int32),
            pltpu.VMEM((6144,), jnp.int32),
            pltpu.VMEM((16,), jnp.int32),
            pltpu.VMEM((1024,), jnp.int32),
            pltpu.VMEM((1024,), jnp.int32),
            pltpu.SemaphoreType.DMA,
        ],
    )
    o = f(sb, vw, thf)
    return [(o[0], o[1]), (o[2], o[3]), (o[4], o[5])]


def _sc_d_body(S0, I0, S1, I1, S2, I2, lab_hbm,
               oa0, oy0, ox0, ot00, ot01, ot02, ot03, ot04,
               oa1, oy1, ox1, ot10, ot11, ot12, ot13, ot14,
               oa2, oy2, ox2, ot20, ot21, ot22, ot23, ot24,
               sbuf, ibuf, idx2d, u0b, u1b, u2b, u3b, u4b,
               ab, yb, xb, t0b, t1b, t2b, t3b, t4b, sem):
    wid = lax.axis_index("s") * 2 + lax.axis_index("c")
    ins = ((S0, I0), (S1, I1), (S2, I2))
    outs = ((oa0, oy0, ox0, ot00, ot01, ot02, ot03, ot04),
            (oa1, oy1, ox1, ot10, ot11, ot12, ot13, ot14),
            (oa2, oy2, ox2, ot20, ot21, ot22, ot23, ot24))
    ubs = (u0b, u1b, u2b, u3b, u4b)
    for s in range(3):
        K = MAX_N_LABELS[s]
        cap = K // NT
        fs = float(FEATURE_SIZES[s])
        Sin, Iin = ins[s]
        pltpu.sync_copy(Sin.at[pl.ds(wid * cap, cap)], sbuf.at[pl.ds(0, cap)])
        pltpu.sync_copy(Iin.at[pl.ds(wid * cap, cap)], ibuf.at[pl.ds(0, cap)])

        def rowidx_body(j, acc):
            off = pl.multiple_of(j * 16, 16)
            iv = ibuf[pl.ds(off, 16)]
            sv = sbuf[pl.ds(off, 16)]
            nz = sv > 0
            ivz = jnp.where(nz, iv, 0)
            v = ivz // NB
            base = ivz - v * NB
            bb = base // 3000
            r1 = base - bb * 3000
            nn = r1 // 3
            row = bb * 1000 + nn
            idx2d[pl.ds(off, 16)] = jnp.where(nz, row, 0)
            return acc

        lax.fori_loop(0, cap // 16, rowidx_body, jnp.int32(0))
        for c in range(5):
            for j2 in range(cap // 128):
                pltpu.async_copy(
                    lab_hbm.at[c].at[idx2d.at[pl.ds(j2 * 128, 128)]],
                    ubs[c].at[pl.ds(j2 * 128, 128)], sem).wait()

        def out_body(j, acc, s=s, fs=fs):
            off = pl.multiple_of(j * 16, 16)
            iv = ibuf[pl.ds(off, 16)]
            sv = sbuf[pl.ds(off, 16)]
            nz = sv > 0
            ivz = jnp.where(nz, iv, 0)
            v = ivz // NB
            base = ivz - v * NB
            bb = base // 3000
            r1 = base - bb * 3000
            nn = r1 // 3
            aa = r1 - nn * 3
            zf = jnp.float32(0.0)
            c = jnp.where(nz, u0b[pl.ds(off, 16)], zf)
            x = jnp.where(nz, u1b[pl.ds(off, 16)] * fs, zf)
            y = jnp.where(nz, u2b[pl.ds(off, 16)] * fs, zf)
            w = jnp.where(nz, u3b[pl.ds(off, 16)] * fs, zf)
            h = jnp.where(nz, u4b[pl.ds(off, 16)] * fs, zf)
            oxv = jnp.where(v == 1, 0.5, jnp.where(v == 3, -0.5, 0.0)).astype(jnp.float32)
            oyv = jnp.where(v == 2, 0.5, jnp.where(v == 4, -0.5, 0.0)).astype(jnp.float32)
            oxv = jnp.where(nz, oxv, zf)
            oyv = jnp.where(nz, oyv, zf)
            xi = jnp.where(x != 0.0, (x - oxv).astype(jnp.int32), 0)
            yi = jnp.where(y != 0.0, (y - oyv).astype(jnp.int32), 0)
            fsi = int(fs)
            ab[pl.ds(off, 16)] = jnp.minimum(jnp.maximum(jnp.where(nz, aa, 0), 0), 4)
            yb[pl.ds(off, 16)] = jnp.minimum(jnp.maximum(yi, 0), fsi - 1)
            xb[pl.ds(off, 16)] = jnp.minimum(jnp.maximum(xi, 0), fsi - 1)
            t0b[pl.ds(off, 16)] = c
            t1b[pl.ds(off, 16)] = x - xi.astype(jnp.float32)
            t2b[pl.ds(off, 16)] = y - yi.astype(jnp.float32)
            t3b[pl.ds(off, 16)] = w
            t4b[pl.ds(off, 16)] = h
            return acc

        lax.fori_loop(0, cap // 16, out_body, jnp.int32(0))
        oo = outs[s]
        for buf, dst in zip((ab, yb, xb, t0b, t1b, t2b, t3b, t4b), oo):
            pltpu.sync_copy(buf.at[pl.ds(0, cap)], dst.at[pl.ds(wid * cap, cap)])


def _run_sc_d(sorted_comp, lab_planes):
    ins = []
    for s in range(3):
        ins += [sorted_comp[s][0], sorted_comp[s][1]]
    shapes = []
    for s in range(3):
        K = MAX_N_LABELS[s]
        shapes += [jax.ShapeDtypeStruct((K,), jnp.int32)] * 3 \
            + [jax.ShapeDtypeStruct((K,), jnp.float32)] * 5
    f = pl.kernel(
        _sc_d_body,
        out_type=tuple(shapes),
        mesh=_sc_mesh(),
        scratch_types=[
            pltpu.VMEM((512,), jnp.int32),
            pltpu.VMEM((512,), jnp.int32),
            pltpu.VMEM((512,), jnp.int32),
            pltpu.VMEM((512,), jnp.float32),
            pltpu.VMEM((512,), jnp.float32),
            pltpu.VMEM((512,), jnp.float32),
            pltpu.VMEM((512,), jnp.float32),
            pltpu.VMEM((512,), jnp.float32),
            pltpu.VMEM((512,), jnp.int32),
            pltpu.VMEM((512,), jnp.int32),
            pltpu.VMEM((512,), jnp.int32),
            pltpu.VMEM((512,), jnp.float32),
            pltpu.VMEM((512,), jnp.float32),
            pltpu.VMEM((512,), jnp.float32),
            pltpu.VMEM((512,), jnp.float32),
            pltpu.VMEM((512,), jnp.float32),
            pltpu.SemaphoreType.DMA,
        ],
    )
    o = f(*ins, lab_planes)
    res = []
    for s in range(3):
        g = o[8 * s: 8 * s + 8]
        tb = jnp.stack(g[3:8], axis=-1)
        res.append((g[0], g[1], g[2], tb))
    return res


def _emu_sc_b(sbits, vmw, th):
    out = []
    sb = sbits.reshape(3, 3 * B * NPAD)
    vw = vmw.reshape(3, 3 * B * NPAD)
    pos0 = jnp.arange(3 * B * NPAD, dtype=jnp.int32)
    a = pos0 >> 16
    bb = (pos0 >> 10) & 63
    nn = pos0 & 1023
    bidx = (bb * 1000 + nn) * 3 + a
    for s in range(3):
        t = th[s, 0]
        bits = sb[s]
        vm = vw[s]
        sel = (bits >= t) & (bits > 0)
        masks = [sel]
        for v in range(1, 5):
            masks.append(sel & (((vm >> (v - 1)) & 1) == 1))
        m = jnp.stack(masks, 0).reshape(-1)
        vals_i = jnp.stack([v * NB + bidx for v in range(5)], 0).reshape(-1)
        vals_s = jnp.stack([bits] * 5, 0).reshape(-1)
        order = jnp.argsort(jnp.where(m, 0, 1).astype(jnp.int32),
                            stable=True)[:CAPS[s]]
        keep = m[order]
        outS = jnp.where(keep, vals_s[order], -1)
        outI = jnp.where(keep, vals_i[order], 0x7FFFFFFF)
        out.append((outS, outI))
    return out


def _emu_sc_d(sorted_comp, labels_flat):
    outs = []
    for s in range(3):
        K = MAX_N_LABELS[s]
        fs = float(FEATURE_SIZES[s])
        sb, ci = sorted_comp[s]
        sb = sb[:K]
        ci = ci[:K]
        nz = sb > 0
        v = jnp.where(nz, ci // NB, 0)
        base = jnp.where(nz, ci % NB, 0)
        bb = base // 3000
        r1 = base - bb * 3000
        nn = r1 // 3
        aa = r1 - nn * 3
        rows = labels_flat[bb * 1000 + nn]
        c = jnp.where(nz, rows[:, 0], 0.0)
        x = jnp.where(nz, rows[:, 1] * fs, 0.0)
        y = jnp.where(nz, rows[:, 2] * fs, 0.0)
        w = jnp.where(nz, rows[:, 3] * fs, 0.0)
        h = jnp.where(nz, rows[:, 4] * fs, 0.0)
        ox = jnp.where(v == 1, 0.5, jnp.where(v == 3, -0.5, 0.0))
        oy = jnp.where(v == 2, 0.5, jnp.where(v == 4, -0.5, 0.0))
        ox = jnp.where(nz, ox, 0.0)
        oy = jnp.where(nz, oy, 0.0)
        xi = jnp.where(x != 0.0, (x - ox).astype(jnp.int32), 0)
        yi = jnp.where(y != 0.0, (y - oy).astype(jnp.int32), 0)
        x_ind = jnp.clip(xi, 0, FEATURE_SIZES[s] - 1)
        y_ind = jnp.clip(yi, 0, FEATURE_SIZES[s] - 1)
        anchor = jnp.where(nz, aa, 0).astype(jnp.int32)
        tb = jnp.stack([c, x - xi.astype(jnp.float32),
                        y - yi.astype(jnp.float32), w, h], axis=-1)
        outs.append((anchor, y_ind, x_ind, tb))
    return outs


_USE_SC = False
_USE_SC_D = False
_INTERP = False


def kernel(real_labels):
    labs_t = jnp.pad(jnp.transpose(real_labels, (2, 0, 1)),
                     ((0, 0), (0, 0), (0, NPAD - N)))
    sbits, vmw, th = _run_tc_a(labs_t, interpret=_INTERP)
    if _USE_SC:
        comp = _run_sc_b(sbits, vmw, th.reshape(1024))
    else:
        comp = _emu_sc_b(sbits, vmw, th)
    sorted_comp = _run_tc_c(comp, interpret=_INTERP)
    labels_flat = real_labels.reshape(B * N, 5)
    if _USE_SC and _USE_SC_D:
        outs = _run_sc_d(sorted_comp, labels_flat.T)
    else:
        outs = _emu_sc_d(sorted_comp, labels_flat)
    return tuple(outs)


# speedup vs baseline: 19.8748x; 2.7793x over previous
import functools

import jax
import jax.numpy as jnp
import numpy as np
from jax import lax
from jax.experimental import pallas as pl
from jax.experimental.pallas import tpu as pltpu
from jax.experimental.pallas import tpu_sc as plsc

STRIDES = [8.0, 16.0, 32.0]
IMAGE_SIZE = 640
FEATURE_SIZES = [int(IMAGE_SIZE / s) for s in STRIDES]
MAX_N_LABELS = [16384, 8192, 4096]
ANCHOR_THRESHOLD = 4.0
HALF_MAX = 65504.0
_ANCHOR_W = [[12.0, 19.0, 40.0], [36.0, 76.0, 72.0], [142.0, 192.0, 459.0]]
_ANCHOR_H = [[16.0, 36.0, 28.0], [75.0, 55.0, 146.0], [110.0, 243.0, 401.0]]
ANCHORS_NP = [np.stack([np.array(_ANCHOR_W[i], dtype=np.float32),
                        np.array(_ANCHOR_H[i], dtype=np.float32)], axis=1)
              / np.float32(STRIDES[i]) for i in range(3)]

B = 64
N = 1000
NPAD = 1024
NB = B * N * 3
CAPS = [32768, 16384, 8192]
NT = 32


def _tc_a_kernel(labs_ref, sbits_ref, vmw_ref, th_ref):
    c = labs_ref[0]
    u1 = labs_ref[1]
    u2 = labs_ref[2]
    u3 = labs_ref[3]
    u4 = labs_ref[4]
    g = 0.5
    for s in range(3):
        fs = float(FEATURE_SIZES[s])
        x = u1 * fs
        y = u2 * fs
        w = u3 * fs
        h = u4 * fs
        w0 = w[0:1, :]
        h0 = h[0:1, :]
        for a in range(3):
            aw = float(ANCHORS_NP[s][a, 0])
            ah = float(ANCHORS_NP[s][a, 1])
            rw = w0 / aw
            rh = h0 / ah
            worse = jnp.maximum(jnp.maximum(rw, 1.0 / rw),
                                jnp.maximum(rh, 1.0 / rh))
            worse = jnp.where(worse != 0.0, worse, HALF_MAX)
            mask = worse < ANCHOR_THRESHOLD
            maskb = jnp.broadcast_to(mask, (B, NPAD))
            aid = float(a + 1)
            score = ((((c + x) + y) + w) + h) + aid
            score = jnp.where(maskb, score, 0.0)
            bx = jnp.where(maskb, x, 0.0)
            by = jnp.where(maskb, y, 0.0)
            ibx = jnp.where(bx != 0.0, fs - bx, 0.0)
            iby = jnp.where(by != 0.0, fs - by, 0.0)
            xm = (jnp.mod(bx, 1.0) < g) & (bx > 1.0)
            ym = (jnp.mod(by, 1.0) < g) & (by > 1.0)
            ixm = (jnp.mod(ibx, 1.0) < g) & (ibx > 1.0)
            iym = (jnp.mod(iby, 1.0) < g) & (iby > 1.0)
            vm = (xm.astype(jnp.int32) | (ym.astype(jnp.int32) << 1)
                  | (ixm.astype(jnp.int32) << 2) | (iym.astype(jnp.int32) << 3))
            cnt = (1 + xm.astype(jnp.int32) + ym.astype(jnp.int32)
                   + ixm.astype(jnp.int32) + iym.astype(jnp.int32))
            sbits_ref[s * 3 + a] = lax.bitcast_convert_type(score, jnp.int32)
            vmw_ref[s * 3 + a] = vm | (cnt << 8)

    ths = []
    for s in range(3):
        bits3 = [sbits_ref[s * 3 + a] for a in range(3)]
        cnt3 = [(vmw_ref[s * 3 + a] >> 8) for a in range(3)]
        K = MAX_N_LABELS[s]

        def step(i, carry, bits3=bits3, cnt3=cnt3, K=K):
            p, kr = carry
            pos = 30 - i
            want = (p << 1) | 1
            c1 = jnp.int32(0)
            for bb, cc in zip(bits3, cnt3):
                m = lax.shift_right_logical(bb, pos) == want
                c1 = c1 + jnp.sum(jnp.where(m, cc, 0))
            take_hi = c1 >= kr
            p = jnp.where(take_hi, want, p << 1)
            kr = jnp.where(take_hi, kr, kr - c1)
            return p, kr

        p, _ = lax.fori_loop(0, 31, step, (jnp.int32(0), jnp.int32(K)))
        ths.append(p)
    rowi = lax.broadcasted_iota(jnp.int32, (8, 128), 0)
    th_ref[...] = jnp.where(rowi == 0, ths[0],
                            jnp.where(rowi == 1, ths[1], ths[2]))


def _run_tc_a(labs_t, interpret=False):
    return pl.pallas_call(
        _tc_a_kernel,
        out_shape=(
            jax.ShapeDtypeStruct((9, B, NPAD), jnp.int32),
            jax.ShapeDtypeStruct((9, B, NPAD), jnp.int32),
            jax.ShapeDtypeStruct((8, 128), jnp.int32),
        ),
        interpret=interpret,
    )(labs_t)


def _partner_rows(x, m):
    r = x.shape[0]
    y = x.reshape(r // (2 * m), 2, m, 128)
    y = jnp.concatenate([y[:, 1:2], y[:, 0:1]], axis=1)
    return y.reshape(r, 128)


def _partner_lanes(x, j):
    lane = lax.broadcasted_iota(jnp.int32, x.shape, 1)
    lo = pltpu.roll(x, 128 - j, 1)
    hi = pltpu.roll(x, j, 1)
    return jnp.where((lane & j) == 0, lo, hi)


def _bitonic_pair(k1, k2, n):
    rows = n // 128
    e_row = lax.broadcasted_iota(jnp.int32, (rows, 128), 0)
    e_lane = lax.broadcasted_iota(jnp.int32, (rows, 128), 1)
    logn = n.bit_length() - 1
    for kl in range(1, logn + 1):
        k = 1 << kl
        if k < 128:
            asc = (e_lane & k) == 0
        elif k < n:
            asc = (e_row & (k // 128)) == 0
        else:
            asc = (e_lane & 0) == 0
        for jl in range(kl - 1, -1, -1):
            j = 1 << jl
            if j < 128:
                p1 = _partner_lanes(k1, j)
                p2 = _partner_lanes(k2, j)
                upper = (e_lane & j) != 0
            else:
                p1 = _partner_rows(k1, j // 128)
                p2 = _partner_rows(k2, j // 128)
                upper = (e_row & (j // 128)) != 0
            less = (p1 < k1) | ((p1 == k1) & (p2 < k2))
            take = less == (asc != upper)
            k1 = jnp.where(take, p1, k1)
            k2 = jnp.where(take, p2, k2)
    return k1, k2


def _tc_c_kernel(s0, i0, s1, i1, s2, i2, os0, oi0, os1, oi1, os2, oi2):
    for (si, ii, oo_s, oo_i, cap) in ((s0, i0, os0, oi0, CAPS[0]),
                                      (s1, i1, os1, oi1, CAPS[1]),
                                      (s2, i2, os2, oi2, CAPS[2])):
        k1 = ~si[...]
        k2 = ii[...]
        k1, k2 = _bitonic_pair(k1, k2, cap)
        oo_s[...] = ~k1
        oo_i[...] = k2


def _run_tc_c(comp, interpret=False):
    args = []
    shapes = []
    for s in range(3):
        cs, ci = comp[s]
        args += [cs.reshape(CAPS[s] // 128, 128), ci.reshape(CAPS[s] // 128, 128)]
        shapes += [jax.ShapeDtypeStruct((CAPS[s] // 128, 128), jnp.int32)] * 2
    outs = pl.pallas_call(
        _tc_c_kernel,
        out_shape=tuple(shapes),
        interpret=interpret,
    )(*args)
    return [(outs[2 * s].reshape(-1), outs[2 * s + 1].reshape(-1))
            for s in range(3)]


def _sc_mesh():
    return plsc.VectorSubcoreMesh(core_axis_name="c", subcore_axis_name="s")
_BIG = jnp.int32(0x7FFFFFFF)


def _sc_b_body(sb_hbm, vw_hbm, th_hbm,
               oS0, oI0, oS1, oI1, oS2, oI2,
               sbuf, vbuf, thv, bufS, bufI, sem):
    del sem
    wid = lax.axis_index("s") * 2 + lax.axis_index("c")
    lanes = lax.iota(jnp.int32, 16)
    outs = ((oS0, oI0), (oS1, oI1), (oS2, oI2))
    for s in range(3):
        cap = CAPS[s] // NT
        pltpu.sync_copy(th_hbm.at[pl.ds(s * 128, 16)], thv)
        tvec = thv[...]
        pltpu.sync_copy(sb_hbm.at[s, wid], sbuf)
        pltpu.sync_copy(vw_hbm.at[s, wid], vbuf)

        def pad_body(j, _, bufS=bufS, bufI=bufI):
            off = pl.multiple_of(j * 16, 16)
            bufS[pl.ds(off, 16)] = jnp.full((16,), -1, jnp.int32)
            bufI[pl.ds(off, 16)] = jnp.full((16,), _BIG, jnp.int32)
            return 0

        lax.fori_loop(0, cap // 16, pad_body, 0)

        def body(i, cur, s=s, cap=cap, tvec=tvec):
            off = pl.multiple_of(i * 16, 16)
            sv = sbuf[pl.ds(off, 16)]
            vv = vbuf[pl.ds(off, 16)]
            sel = (sv >= tvec) & (sv > 0)
            posg = (wid * 6144 + i * 16) + lanes
            a = lax.shift_right_logical(posg, 16)
            b = lax.shift_right_logical(posg, 10) & 63
            n = posg & 1023
            bidx = (b * 1000 + n) * 3 + a
            for v in range(5):
                if v == 0:
                    mv = sel
                else:
                    mv = sel & ((lax.shift_right_logical(vv, v - 1) & 1) == 1)
                mi = mv.astype(jnp.int32)
                cs = plsc.cumsum(mi)
                pos = (cur + cs) - mi
                okm = mv & (pos < cap)
                plsc.store_scatter(bufS, [pos], sv, mask=okm)
                plsc.store_scatter(bufI, [pos], v * NB + bidx, mask=okm)
                cur = cur + plsc.cummax(lax.rev(cs, (0,)))
            return cur

        lax.fori_loop(0, 6144 // 16, body, jnp.zeros((16,), jnp.int32))
        oS, oI = outs[s]
        pltpu.sync_copy(bufS.at[pl.ds(0, cap)], oS.at[pl.ds(wid * cap, cap)])
        pltpu.sync_copy(bufI.at[pl.ds(0, cap)], oI.at[pl.ds(wid * cap, cap)])


def _run_sc_b(sbits, vmw, th):
    sb = sbits.reshape(3, NT, 6144)
    vw = vmw.reshape(3, NT, 6144)
    thf = th.reshape(1024)
    f = pl.kernel(
        _sc_b_body,
        out_type=tuple(jax.ShapeDtypeStruct((CAPS[s // 2],), jnp.int32)
                       for s in range(6)),
        mesh=_sc_mesh(),
        scratch_types=[
            pltpu.VMEM((6144,), jnp.int32),
            pltpu.VMEM((6144,), jnp.int32),
            pltpu.VMEM((16,), jnp.int32),
            pltpu.VMEM((1024,), jnp.int32),
            pltpu.VMEM((1024,), jnp.int32),
            pltpu.SemaphoreType.DMA,
        ],
    )
    o = f(sb, vw, thf)
    return [(o[0], o[1]), (o[2], o[3]), (o[4], o[5])]


def _sc_d_body(S0, I0, S1, I1, S2, I2, lab_hbm,
               oa0, oy0, ox0, ot00, ot01, ot02, ot03, ot04,
               oa1, oy1, ox1, ot10, ot11, ot12, ot13, ot14,
               oa2, oy2, ox2, ot20, ot21, ot22, ot23, ot24,
               sbuf, ibuf, idx2d, u0b, u1b, u2b, u3b, u4b,
               ab, yb, xb, t0b, t1b, t2b, t3b, t4b, sem):
    wid = lax.axis_index("s") * 2 + lax.axis_index("c")
    ins = ((S0, I0), (S1, I1), (S2, I2))
    outs = ((oa0, oy0, ox0, ot00, ot01, ot02, ot03, ot04),
            (oa1, oy1, ox1, ot10, ot11, ot12, ot13, ot14),
            (oa2, oy2, ox2, ot20, ot21, ot22, ot23, ot24))
    ubs = (u0b, u1b, u2b, u3b, u4b)
    for s in range(3):
        K = MAX_N_LABELS[s]
        cap = K // NT
        fs = float(FEATURE_SIZES[s])
        Sin, Iin = ins[s]
        pltpu.sync_copy(Sin.at[pl.ds(wid * cap, cap)], sbuf.at[pl.ds(0, cap)])
        pltpu.sync_copy(Iin.at[pl.ds(wid * cap, cap)], ibuf.at[pl.ds(0, cap)])

        def rowidx_body(j, acc):
            off = pl.multiple_of(j * 16, 16)
            iv = ibuf[pl.ds(off, 16)]
            sv = sbuf[pl.ds(off, 16)]
            nz = sv > 0
            ivz = jnp.where(nz, iv, 0)
            v = ivz // NB
            base = ivz - v * NB
            bb = base // 3000
            r1 = base - bb * 3000
            nn = r1 // 3
            row = bb * 1000 + nn
            idx2d[pl.ds(off, 16)] = jnp.where(nz, row, 0)
            return acc

        lax.fori_loop(0, cap // 16, rowidx_body, jnp.int32(0))
        for c in range(5):
            for j2 in range(cap // 128):
                pltpu.async_copy(
                    lab_hbm.at[c].at[idx2d.at[pl.ds(j2 * 128, 128)]],
                    ubs[c].at[pl.ds(j2 * 128, 128)], sem).wait()

        def out_body(j, acc, s=s, fs=fs):
            off = pl.multiple_of(j * 16, 16)
            iv = ibuf[pl.ds(off, 16)]
            sv = sbuf[pl.ds(off, 16)]
            nz = sv > 0
            ivz = jnp.where(nz, iv, 0)
            v = ivz // NB
            base = ivz - v * NB
            bb = base // 3000
            r1 = base - bb * 3000
            nn = r1 // 3
            aa = r1 - nn * 3
            zf = jnp.float32(0.0)
            c = jnp.where(nz, u0b[pl.ds(off, 16)], zf)
            x = jnp.where(nz, u1b[pl.ds(off, 16)] * fs, zf)
            y = jnp.where(nz, u2b[pl.ds(off, 16)] * fs, zf)
            w = jnp.where(nz, u3b[pl.ds(off, 16)] * fs, zf)
            h = jnp.where(nz, u4b[pl.ds(off, 16)] * fs, zf)
            oxv = jnp.where(v == 1, 0.5, jnp.where(v == 3, -0.5, 0.0)).astype(jnp.float32)
            oyv = jnp.where(v == 2, 0.5, jnp.where(v == 4, -0.5, 0.0)).astype(jnp.float32)
            oxv = jnp.where(nz, oxv, zf)
            oyv = jnp.where(nz, oyv, zf)
            xi = jnp.where(x != 0.0, (x - oxv).astype(jnp.int32), 0)
            yi = jnp.where(y != 0.0, (y - oyv).astype(jnp.int32), 0)
            fsi = int(fs)
            ab[pl.ds(off, 16)] = jnp.minimum(jnp.maximum(jnp.where(nz, aa, 0), 0), 4)
            yb[pl.ds(off, 16)] = jnp.minimum(jnp.maximum(yi, 0), fsi - 1)
            xb[pl.ds(off, 16)] = jnp.minimum(jnp.maximum(xi, 0), fsi - 1)
            t0b[pl.ds(off, 16)] = c
            t1b[pl.ds(off, 16)] = x - xi.astype(jnp.float32)
            t2b[pl.ds(off, 16)] = y - yi.astype(jnp.float32)
            t3b[pl.ds(off, 16)] = w
            t4b[pl.ds(off, 16)] = h
            return acc

        lax.fori_loop(0, cap // 16, out_body, jnp.int32(0))
        oo = outs[s]
        for buf, dst in zip((ab, yb, xb, t0b, t1b, t2b, t3b, t4b), oo):
            pltpu.sync_copy(buf.at[pl.ds(0, cap)], dst.at[pl.ds(wid * cap, cap)])


def _run_sc_d(sorted_comp, lab_planes):
    ins = []
    for s in range(3):
        ins += [sorted_comp[s][0], sorted_comp[s][1]]
    shapes = []
    for s in range(3):
        K = MAX_N_LABELS[s]
        shapes += [jax.ShapeDtypeStruct((K,), jnp.int32)] * 3 \
            + [jax.ShapeDtypeStruct((K,), jnp.float32)] * 5
    f = pl.kernel(
        _sc_d_body,
        out_type=tuple(shapes),
        mesh=_sc_mesh(),
        scratch_types=[
            pltpu.VMEM((512,), jnp.int32),
            pltpu.VMEM((512,), jnp.int32),
            pltpu.VMEM((512,), jnp.int32),
            pltpu.VMEM((512,), jnp.float32),
            pltpu.VMEM((512,), jnp.float32),
            pltpu.VMEM((512,), jnp.float32),
            pltpu.VMEM((512,), jnp.float32),
            pltpu.VMEM((512,), jnp.float32),
            pltpu.VMEM((512,), jnp.int32),
            pltpu.VMEM((512,), jnp.int32),
            pltpu.VMEM((512,), jnp.int32),
            pltpu.VMEM((512,), jnp.float32),
            pltpu.VMEM((512,), jnp.float32),
            pltpu.VMEM((512,), jnp.float32),
            pltpu.VMEM((512,), jnp.float32),
            pltpu.VMEM((512,), jnp.float32),
            pltpu.SemaphoreType.DMA,
        ],
    )
    o = f(*ins, lab_planes)
    res = []
    for s in range(3):
        g = o[8 * s: 8 * s + 8]
        tb = jnp.stack(g[3:8], axis=-1)
        res.append((g[0], g[1], g[2], tb))
    return res


def _emu_sc_b(sbits, vmw, th):
    out = []
    sb = sbits.reshape(3, 3 * B * NPAD)
    vw = vmw.reshape(3, 3 * B * NPAD)
    pos0 = jnp.arange(3 * B * NPAD, dtype=jnp.int32)
    a = pos0 >> 16
    bb = (pos0 >> 10) & 63
    nn = pos0 & 1023
    bidx = (bb * 1000 + nn) * 3 + a
    for s in range(3):
        t = th[s, 0]
        bits = sb[s]
        vm = vw[s]
        sel = (bits >= t) & (bits > 0)
        cb = MAX_N_LABELS[s] + 1024
        ob = jnp.argsort(jnp.where(sel, 0, 1).astype(jnp.int32),
                         stable=True)[:cb]
        keep_b = sel[ob]
        gbits = bits[ob]
        gvm = vm[ob]
        gbidx = bidx[ob]
        masks = [keep_b]
        for v in range(1, 5):
            masks.append(keep_b & (((gvm >> (v - 1)) & 1) == 1))
        m = jnp.stack(masks, 0).reshape(-1)
        vals_i = jnp.stack([v * NB + gbidx for v in range(5)], 0).reshape(-1)
        vals_s = jnp.stack([gbits] * 5, 0).reshape(-1)
        order = jnp.argsort(jnp.where(m, 0, 1).astype(jnp.int32),
                            stable=True)[:CAPS[s]]
        keep = m[order]
        outS = jnp.where(keep, vals_s[order], -1)
        outI = jnp.where(keep, vals_i[order], 0x7FFFFFFF)
        out.append((outS, outI))
    return out


def _emu_sc_d(sorted_comp, labels_flat):
    outs = []
    for s in range(3):
        K = MAX_N_LABELS[s]
        fs = float(FEATURE_SIZES[s])
        sb, ci = sorted_comp[s]
        sb = sb[:K]
        ci = ci[:K]
        nz = sb > 0
        v = jnp.where(nz, ci // NB, 0)
        base = jnp.where(nz, ci % NB, 0)
        bb = base // 3000
        r1 = base - bb * 3000
        nn = r1 // 3
        aa = r1 - nn * 3
        rows = labels_flat[bb * 1000 + nn]
        c = jnp.where(nz, rows[:, 0], 0.0)
        x = jnp.where(nz, rows[:, 1] * fs, 0.0)
        y = jnp.where(nz, rows[:, 2] * fs, 0.0)
        w = jnp.where(nz, rows[:, 3] * fs, 0.0)
        h = jnp.where(nz, rows[:, 4] * fs, 0.0)
        ox = jnp.where(v == 1, 0.5, jnp.where(v == 3, -0.5, 0.0))
        oy = jnp.where(v == 2, 0.5, jnp.where(v == 4, -0.5, 0.0))
        ox = jnp.where(nz, ox, 0.0)
        oy = jnp.where(nz, oy, 0.0)
        xi = jnp.where(x != 0.0, (x - ox).astype(jnp.int32), 0)
        yi = jnp.where(y != 0.0, (y - oy).astype(jnp.int32), 0)
        x_ind = jnp.clip(xi, 0, FEATURE_SIZES[s] - 1)
        y_ind = jnp.clip(yi, 0, FEATURE_SIZES[s] - 1)
        anchor = jnp.where(nz, aa, 0).astype(jnp.int32)
        tb = jnp.stack([c, x - xi.astype(jnp.float32),
                        y - yi.astype(jnp.float32), w, h], axis=-1)
        outs.append((anchor, y_ind, x_ind, tb))
    return outs


_USE_SC = False
_USE_SC_D = False
_INTERP = False


def kernel(real_labels):
    labs_t = jnp.pad(jnp.transpose(real_labels, (2, 0, 1)),
                     ((0, 0), (0, 0), (0, NPAD - N)))
    sbits, vmw, th = _run_tc_a(labs_t, interpret=_INTERP)
    if _USE_SC:
        comp = _run_sc_b(sbits, vmw, th.reshape(1024))
    else:
        comp = _emu_sc_b(sbits, vmw, th)
    sorted_comp = _run_tc_c(comp, interpret=_INTERP)
    labels_flat = real_labels.reshape(B * N, 5)
    if _USE_SC and _USE_SC_D:
        outs = _run_sc_d(sorted_comp, labels_flat.T)
    else:
        outs = _emu_sc_d(sorted_comp, labels_flat)
    return tuple(outs)
